# baseline (speedup 1.0000x reference)
import functools

import jax
import jax.numpy as jnp
from jax import lax
from jax.experimental import pallas as pl
from jax.experimental.pallas import tpu as pltpu
from jax.experimental.pallas import tpu_sc as plsc

BATCH = 4
SEQ = 2048
DIM = 64
NUM_CORES = 2
NUM_SUBCORES = 16
NUM_WORKERS = NUM_CORES * NUM_SUBCORES
N = BATCH * SEQ
PER_W = N // NUM_WORKERS
IDX_CHUNK = 128
N_CHUNKS = PER_W // IDX_CHUNK
LANES = 16
DCHUNKS = DIM // LANES


def _emb_body(x_hbm, table_hbm, pe_hbm, out_hbm, idx_v, rows_v, pe_v, sem):
    wid = lax.axis_index("s") * NUM_CORES + lax.axis_index("c")
    base = wid * PER_W
    pe_base = base % SEQ

    pltpu.sync_copy(x_hbm.at[pl.ds(base, PER_W)], idx_v)
    pe_cp = pltpu.async_copy(pe_hbm.at[pl.ds(pe_base, PER_W)], pe_v, sem)
    gathers = [
        pltpu.async_copy(
            table_hbm.at[idx_v.at[pl.ds(j * IDX_CHUNK, IDX_CHUNK)]],
            rows_v.at[pl.ds(j * IDX_CHUNK, IDX_CHUNK)],
            sem,
        )
        for j in range(N_CHUNKS)
    ]
    pe_cp.wait()
    for g in gathers:
        g.wait()

    def add_row(i, carry):
        for c in range(DCHUNKS):
            sl = (i, pl.ds(c * LANES, LANES))
            rows_v[sl] = rows_v[sl] + pe_v[sl]
        return carry

    lax.fori_loop(0, PER_W, add_row, 0)
    pltpu.sync_copy(rows_v, out_hbm.at[pl.ds(base, PER_W)])


_emb_call = pl.kernel(
    _emb_body,
    out_type=jax.ShapeDtypeStruct((N, DIM), jnp.float32),
    mesh=plsc.VectorSubcoreMesh(
        core_axis_name="c",
        subcore_axis_name="s",
        num_cores=NUM_CORES,
        num_subcores=NUM_SUBCORES,
    ),
    compiler_params=pltpu.CompilerParams(use_tc_tiling_on_sc=False),
    scratch_types=[
        pltpu.VMEM((PER_W,), jnp.int32),
        pltpu.VMEM((PER_W, DIM), jnp.float32),
        pltpu.VMEM((PER_W, DIM), jnp.float32),
        pltpu.SemaphoreType.DMA,
    ],
)


@jax.jit
def kernel(x, table, pe):
    out = _emb_call(x.reshape(N), table, pe)
    return out.reshape(BATCH, SEQ, DIM)


# speedup vs baseline: 4.4791x; 4.4791x over previous
import jax
import jax.numpy as jnp
from jax import lax
from jax.experimental import pallas as pl
from jax.experimental.pallas import tpu as pltpu
from jax.experimental.pallas import tpu_sc as plsc

NC, NS = 2, 16
PER_W = 256
GROUP = 16
N_GROUPS = PER_W // GROUP
NBUF = 4


def body(x_hbm, tt_hbm, pe_hbm, out_hbm, idx_smem, idx_v, pe_v, blk, g_v, sems, pe_sem):
    wid = lax.axis_index("s") * NC + lax.axis_index("c")
    base = wid * PER_W
    pe_base = base % 2048

    pltpu.sync_copy(x_hbm.at[pl.ds(base, PER_W)], idx_v)
    pe_cp = pltpu.async_copy(pe_hbm.at[pl.ds(pe_base, PER_W)], pe_v, pe_sem)

    def spill(g, carry):
        vt = idx_v[pl.ds(g * GROUP, GROUP)]
        for j in range(GROUP):
            idx_smem[g * GROUP + j] = vt[j]
        return carry

    lax.fori_loop(0, N_GROUPS, spill, 0)

    def fire(t, slot):
        i = idx_smem[t]
        off = pl.multiple_of((i >> 7) * 128, 128)
        soff = pl.multiple_of(slot * 64, 64)
        pltpu.async_copy(
            tt_hbm.at[:, pl.ds(off, 128)],
            blk.at[pl.ds(soff, 64)],
            sems.at[slot],
        )

    def extract(t, slot):
        i = idx_smem[t]
        soff = pl.multiple_of(slot * 64, 64)
        pltpu.make_async_copy(
            tt_hbm.at[:, pl.ds(0, 128)],
            blk.at[pl.ds(soff, 64)],
            sems.at[slot],
        ).wait()
        cvec = jnp.zeros((16,), jnp.int32) + (i & 127)
        dvec = lax.iota(jnp.int32, 16)
        for k in range(4):
            vals = plsc.load_gather(blk, [soff + dvec + k * 16, cvec])
            g_v[t, pl.ds(k * 16, 16)] = vals

    def prologue(s, carry):
        fire(s, s)
        return carry

    lax.fori_loop(0, NBUF, prologue, 0)

    def steady(tl, carry):
        slot = lax.rem(tl, NBUF)
        extract(tl, slot)

        @pl.when(tl < PER_W - NBUF)
        def _():
            fire(tl + NBUF, slot)

        return carry

    lax.fori_loop(0, PER_W, steady, 0)

    pe_cp.wait()

    def add_row(t, carry):
        for c in range(4):
            sl = (t, pl.ds(c * 16, 16))
            g_v[sl] = g_v[sl] + pe_v[sl]
        return carry

    lax.fori_loop(0, PER_W, add_row, 0)
    pltpu.sync_copy(g_v, out_hbm.at[pl.ds(base, PER_W)])


call = pl.kernel(
    body,
    out_type=jax.ShapeDtypeStruct((8192, 64), jnp.float32),
    mesh=plsc.VectorSubcoreMesh(
        core_axis_name="c", subcore_axis_name="s", num_cores=NC, num_subcores=NS
    ),
    compiler_params=pltpu.CompilerParams(needs_layout_passes=False),
    scratch_types=[
        pltpu.SMEM((PER_W,), jnp.int32),
        pltpu.VMEM((PER_W,), jnp.int32),
        pltpu.VMEM((PER_W, 64), jnp.float32),
        pltpu.VMEM((NBUF * 64, 128), jnp.float32),
        pltpu.VMEM((PER_W, 64), jnp.float32),
        pltpu.SemaphoreType.DMA((NBUF,)),
        pltpu.SemaphoreType.DMA,
    ],
)


@jax.jit
def kernel(x, table, pe):
    out = call(x.reshape(8192), table.T, pe)
    return out.reshape(4, 2048, 64)
